# baseline (speedup 1.0000x reference)
import jax
import jax.numpy as jnp
from jax import lax
from jax.experimental import pallas as pl
from jax.experimental.pallas import tpu as pltpu
from jax.experimental.pallas import tpu_sc as plsc

_N_EMB = 1024
_DIM = 64
_B = 16
_T = 1024
_TOKENS = _B * _T
_BM = 1024
_G = _TOKENS // _BM
_NL = _N_EMB // 128
_NW = 32
_BPW = _TOKENS // _NW
_CHUNK = 128
_NCH = _BPW // _CHUNK
_LOSS_SCALE = 1.25 / float(_TOKENS * _DIM)


def _dist_argmin_kernel(x_ref, e_ref, idx_ref, loss_ref):
    x = x_ref[0]
    e = e_ref[...]
    mm = lax.dot_general(x, e, (((1,), (1,)), ((), ())),
                         preferred_element_type=jnp.float32)
    x2 = jnp.sum(x * x, axis=1, keepdims=True)
    e2 = jnp.sum(e * e, axis=1)
    dist = (x2 - 2.0 * mm) + e2[None, :]
    m = jnp.min(dist, axis=1, keepdims=True)

    ids = lax.broadcasted_iota(jnp.int32, dist.shape, 1)
    idx = jnp.min(jnp.where(dist == m, ids, jnp.int32(_N_EMB)), axis=1)
    idx_ref[0, 0, :] = idx

    @pl.when(pl.program_id(0) == 0)
    def _init():
        loss_ref[...] = jnp.zeros((1, 1), jnp.float32)

    loss_ref[...] += jnp.sum(m)[None, None]

    @pl.when(pl.program_id(0) == pl.num_programs(0) - 1)
    def _finalize():
        loss_ref[...] = loss_ref[...] * _LOSS_SCALE


_dist_call = pl.pallas_call(
    _dist_argmin_kernel,
    grid=(_G,),
    in_specs=[
        pl.BlockSpec((1, _BM, _DIM), lambda i: (i, 0, 0)),
        pl.BlockSpec((_N_EMB, _DIM), lambda i: (0, 0)),
    ],
    out_specs=[
        pl.BlockSpec((1, 1, _BM), lambda i: (i, 0, 0)),
        pl.BlockSpec((1, 1), lambda i: (0, 0)),
    ],
    out_shape=[
        jax.ShapeDtypeStruct((_G, 1, _BM), jnp.int32),
        jax.ShapeDtypeStruct((1, 1), jnp.float32),
    ],
)


def _gather_body(idx_hbm, table_hbm, out_hbm, idx_v, rows_v, sem):
    wid = lax.axis_index("s") * 2 + lax.axis_index("c")
    pltpu.sync_copy(idx_hbm.at[wid], idx_v)
    for j in range(_NCH):
        pltpu.async_copy(table_hbm.at[idx_v.at[j]],
                         rows_v.at[pl.ds(j * _CHUNK, _CHUNK)], sem).wait()
    g, h = wid // 2, wid % 2
    pltpu.sync_copy(rows_v, out_hbm.at[g, pl.ds(h * _BPW, _BPW)])


_gather_call = pl.kernel(
    _gather_body,
    out_type=jax.ShapeDtypeStruct((_B, _T, _DIM), jnp.float32),
    mesh=plsc.VectorSubcoreMesh(core_axis_name="c", subcore_axis_name="s"),
    compiler_params=pltpu.CompilerParams(use_tc_tiling_on_sc=False),
    scratch_types=[
        pltpu.VMEM((_NCH, _CHUNK), jnp.int32),
        pltpu.VMEM((_BPW, _DIM), jnp.float32),
        pltpu.SemaphoreType.DMA,
    ],
)


@jax.jit
def kernel(inputs, embedding):
    idx3, loss = _dist_call(inputs, embedding)
    indices = idx3.reshape(_TOKENS)
    quantized_st = _gather_call(indices.reshape(_NW, _NCH, _CHUNK), embedding)
    return quantized_st, loss[0, 0], indices


# speedup vs baseline: 1.0878x; 1.0878x over previous
import jax
import jax.numpy as jnp
from jax import lax
from jax.experimental import pallas as pl
from jax.experimental.pallas import tpu as pltpu
from jax.experimental.pallas import tpu_sc as plsc

_N_EMB = 1024
_DIM = 64
_B = 16
_T = 1024
_TOKENS = _B * _T
_BM = 2048
_G = _TOKENS // _BM
_NL = _N_EMB // 128
_NW = 32
_BPW = _TOKENS // _NW
_CHUNK = 128
_NCH = _BPW // _CHUNK
_LOSS_SCALE = 1.25 / float(_TOKENS * _DIM)


def _dist_argmin_kernel(x_ref, e_ref, idx_ref, loss_ref):
    x = x_ref[...].reshape(_BM, _DIM)
    e = e_ref[...]
    mm = lax.dot_general(x, e, (((1,), (1,)), ((), ())),
                         preferred_element_type=jnp.float32)
    x2 = jnp.sum(x * x, axis=1, keepdims=True)
    e2 = jnp.sum(e * e, axis=1)
    dist = (x2 - 2.0 * mm) + e2[None, :]
    m = jnp.min(dist, axis=1, keepdims=True)

    ids = lax.broadcasted_iota(jnp.int32, dist.shape, 1).astype(jnp.float32)
    idx = jnp.min(jnp.where(dist == m, ids, jnp.float32(_N_EMB)), axis=1)
    idx_ref[0, 0, :] = idx.astype(jnp.int32)

    @pl.when(pl.program_id(0) == 0)
    def _init():
        loss_ref[...] = jnp.zeros((1, 1), jnp.float32)

    loss_ref[...] += jnp.sum(m)[None, None]

    @pl.when(pl.program_id(0) == pl.num_programs(0) - 1)
    def _finalize():
        loss_ref[...] = loss_ref[...] * _LOSS_SCALE


_dist_call = pl.pallas_call(
    _dist_argmin_kernel,
    grid=(_G,),
    in_specs=[
        pl.BlockSpec((_BM // _T, _T, _DIM), lambda i: (i, 0, 0)),
        pl.BlockSpec((_N_EMB, _DIM), lambda i: (0, 0)),
    ],
    out_specs=[
        pl.BlockSpec((1, 1, _BM), lambda i: (i, 0, 0)),
        pl.BlockSpec((1, 1), lambda i: (0, 0)),
    ],
    out_shape=[
        jax.ShapeDtypeStruct((_G, 1, _BM), jnp.int32),
        jax.ShapeDtypeStruct((1, 1), jnp.float32),
    ],
)


def _gather_body(idx_hbm, table_hbm, out_hbm, idx_v, rows_v, sem):
    wid = lax.axis_index("s") * 2 + lax.axis_index("c")
    pltpu.sync_copy(idx_hbm.at[wid], idx_v)
    for j in range(_NCH):
        pltpu.async_copy(table_hbm.at[idx_v.at[j]],
                         rows_v.at[pl.ds(j * _CHUNK, _CHUNK)], sem).wait()
    g, h = wid // 2, wid % 2
    pltpu.sync_copy(rows_v, out_hbm.at[g, pl.ds(h * _BPW, _BPW)])


_gather_call = pl.kernel(
    _gather_body,
    out_type=jax.ShapeDtypeStruct((_B, _T, _DIM), jnp.float32),
    mesh=plsc.VectorSubcoreMesh(core_axis_name="c", subcore_axis_name="s"),
    compiler_params=pltpu.CompilerParams(use_tc_tiling_on_sc=False),
    scratch_types=[
        pltpu.VMEM((_NCH, _CHUNK), jnp.int32),
        pltpu.VMEM((_BPW, _DIM), jnp.float32),
        pltpu.SemaphoreType.DMA,
    ],
)


@jax.jit
def kernel(inputs, embedding):
    idx3, loss = _dist_call(inputs, embedding)
    indices = idx3.reshape(_TOKENS)
    quantized_st = _gather_call(indices.reshape(_NW, _NCH, _CHUNK), embedding)
    return quantized_st, loss[0, 0], indices


# speedup vs baseline: 1.1264x; 1.0354x over previous
import jax
import jax.numpy as jnp
from jax import lax
from jax.experimental import pallas as pl
from jax.experimental.pallas import tpu as pltpu
from jax.experimental.pallas import tpu_sc as plsc

_N_EMB = 1024
_DIM = 64
_B = 16
_T = 1024
_TOKENS = _B * _T
_BM = 4096
_G = _TOKENS // _BM
_NL = _N_EMB // 128
_NW = 32
_BPW = _TOKENS // _NW
_CHUNK = 128
_NCH = _BPW // _CHUNK
_LOSS_SCALE = 1.25 / float(_TOKENS * _DIM)


def _dist_argmin_kernel(x_ref, e_ref, idx_ref, loss_ref):
    x = x_ref[...].reshape(_BM, _DIM)
    e = e_ref[...]
    mm = lax.dot_general(x, e, (((1,), (1,)), ((), ())),
                         preferred_element_type=jnp.float32)
    x2 = jnp.sum(x * x, axis=1, keepdims=True)
    e2 = jnp.sum(e * e, axis=1)
    dist = (x2 - 2.0 * mm) + e2[None, :]
    m = jnp.min(dist, axis=1, keepdims=True)

    ids = lax.broadcasted_iota(jnp.int32, dist.shape, 1).astype(jnp.float32)
    idx = jnp.min(jnp.where(dist == m, ids, jnp.float32(_N_EMB)), axis=1)
    idx_ref[0, 0, :] = idx.astype(jnp.int32)

    @pl.when(pl.program_id(0) == 0)
    def _init():
        loss_ref[...] = jnp.zeros((1, 1), jnp.float32)

    loss_ref[...] += jnp.sum(m)[None, None]

    @pl.when(pl.program_id(0) == pl.num_programs(0) - 1)
    def _finalize():
        loss_ref[...] = loss_ref[...] * _LOSS_SCALE


_dist_call = pl.pallas_call(
    _dist_argmin_kernel,
    grid=(_G,),
    in_specs=[
        pl.BlockSpec((_BM // _T, _T, _DIM), lambda i: (i, 0, 0)),
        pl.BlockSpec((_N_EMB, _DIM), lambda i: (0, 0)),
    ],
    out_specs=[
        pl.BlockSpec((1, 1, _BM), lambda i: (i, 0, 0)),
        pl.BlockSpec((1, 1), lambda i: (0, 0)),
    ],
    out_shape=[
        jax.ShapeDtypeStruct((_G, 1, _BM), jnp.int32),
        jax.ShapeDtypeStruct((1, 1), jnp.float32),
    ],
)


def _gather_body(idx_hbm, table_hbm, out_hbm, idx_v, rows_v, sem):
    wid = lax.axis_index("s") * 2 + lax.axis_index("c")
    pltpu.sync_copy(idx_hbm.at[wid], idx_v)
    copies = [pltpu.async_copy(table_hbm.at[idx_v.at[j]],
                               rows_v.at[pl.ds(j * _CHUNK, _CHUNK)], sem)
              for j in range(_NCH)]
    for c in copies:
        c.wait()
    g, h = wid // 2, wid % 2
    pltpu.sync_copy(rows_v, out_hbm.at[g, pl.ds(h * _BPW, _BPW)])


_gather_call = pl.kernel(
    _gather_body,
    out_type=jax.ShapeDtypeStruct((_B, _T, _DIM), jnp.float32),
    mesh=plsc.VectorSubcoreMesh(core_axis_name="c", subcore_axis_name="s"),
    compiler_params=pltpu.CompilerParams(use_tc_tiling_on_sc=False),
    scratch_types=[
        pltpu.VMEM((_NCH, _CHUNK), jnp.int32),
        pltpu.VMEM((_BPW, _DIM), jnp.float32),
        pltpu.SemaphoreType.DMA,
    ],
)


@jax.jit
def kernel(inputs, embedding):
    idx3, loss = _dist_call(inputs, embedding)
    indices = idx3.reshape(_TOKENS)
    quantized_st = _gather_call(indices.reshape(_NW, _NCH, _CHUNK), embedding)
    return quantized_st, loss[0, 0], indices


# speedup vs baseline: 1.6011x; 1.4215x over previous
import jax
import jax.numpy as jnp
from jax import lax
from jax.experimental import pallas as pl
from jax.experimental.pallas import tpu as pltpu
from jax.experimental.pallas import tpu_sc as plsc

_N_EMB = 1024
_DIM = 64
_B = 16
_T = 1024
_TOKENS = _B * _T
_BM = 4096
_G = _TOKENS // _BM
_NL = _N_EMB // 128
_NW = 32
_BPW = _TOKENS // _NW
_CHUNK = 128
_NCH = _BPW // _CHUNK
_LOSS_SCALE = 1.25 / float(_TOKENS * _DIM)


def _dist_argmin_kernel(x_ref, e_ref, idx_ref, loss_ref):
    e = e_ref[...]
    e2 = jnp.sum(e * e, axis=1)[:, None]
    loss_part = None
    for bb in range(_BM // _T):
        xt = x_ref[bb]
        mm = lax.dot_general(e, xt, (((1,), (0,)), ((), ())),
                             preferred_element_type=jnp.float32)
        x2 = jnp.sum(xt * xt, axis=0)[None, :]
        dist = (x2 - 2.0 * mm) + e2
        m = jnp.min(dist, axis=0, keepdims=True)

        ids = lax.broadcasted_iota(jnp.int32, dist.shape, 0).astype(jnp.float32)
        idx = jnp.min(jnp.where(dist == m, ids, jnp.float32(_N_EMB)), axis=0)
        idx_ref[0, 0, bb * _T:(bb + 1) * _T] = idx.astype(jnp.int32)
        s = jnp.sum(m)
        loss_part = s if loss_part is None else loss_part + s

    @pl.when(pl.program_id(0) == 0)
    def _init():
        loss_ref[...] = jnp.zeros((1, 1), jnp.float32)

    loss_ref[...] += loss_part[None, None]

    @pl.when(pl.program_id(0) == pl.num_programs(0) - 1)
    def _finalize():
        loss_ref[...] = loss_ref[...] * _LOSS_SCALE


_dist_call = pl.pallas_call(
    _dist_argmin_kernel,
    grid=(_G,),
    in_specs=[
        pl.BlockSpec((_BM // _T, _DIM, _T), lambda i: (i, 0, 0)),
        pl.BlockSpec((_N_EMB, _DIM), lambda i: (0, 0)),
    ],
    out_specs=[
        pl.BlockSpec((1, 1, _BM), lambda i: (i, 0, 0)),
        pl.BlockSpec((1, 1), lambda i: (0, 0)),
    ],
    out_shape=[
        jax.ShapeDtypeStruct((_G, 1, _BM), jnp.int32),
        jax.ShapeDtypeStruct((1, 1), jnp.float32),
    ],
)


def _gather_body(idx_hbm, table_hbm, out_hbm, idx_v, rows_v, sem):
    wid = lax.axis_index("s") * 2 + lax.axis_index("c")
    pltpu.sync_copy(idx_hbm.at[wid], idx_v)
    copies = [pltpu.async_copy(table_hbm.at[idx_v.at[j]],
                               rows_v.at[pl.ds(j * _CHUNK, _CHUNK)], sem)
              for j in range(_NCH)]
    for c in copies:
        c.wait()
    g, h = wid // 2, wid % 2
    pltpu.sync_copy(rows_v, out_hbm.at[g, pl.ds(h * _BPW, _BPW)])


_gather_call = pl.kernel(
    _gather_body,
    out_type=jax.ShapeDtypeStruct((_B, _T, _DIM), jnp.float32),
    mesh=plsc.VectorSubcoreMesh(core_axis_name="c", subcore_axis_name="s"),
    compiler_params=pltpu.CompilerParams(use_tc_tiling_on_sc=False),
    scratch_types=[
        pltpu.VMEM((_NCH, _CHUNK), jnp.int32),
        pltpu.VMEM((_BPW, _DIM), jnp.float32),
        pltpu.SemaphoreType.DMA,
    ],
)


@jax.jit
def kernel(inputs, embedding):
    idx3, loss = _dist_call(jnp.swapaxes(inputs, 1, 2), embedding)
    indices = idx3.reshape(_TOKENS)
    quantized_st = _gather_call(indices.reshape(_NW, _NCH, _CHUNK), embedding)
    return quantized_st, loss[0, 0], indices


# speedup vs baseline: 1.6130x; 1.0074x over previous
import jax
import jax.numpy as jnp
from jax import lax
from jax.experimental import pallas as pl
from jax.experimental.pallas import tpu as pltpu
from jax.experimental.pallas import tpu_sc as plsc

_N_EMB = 1024
_DIM = 64
_B = 16
_T = 1024
_TOKENS = _B * _T
_BM = 8192
_G = _TOKENS // _BM
_NL = _N_EMB // 128
_NW = 32
_BPW = _TOKENS // _NW
_CHUNK = 128
_NCH = _BPW // _CHUNK
_LOSS_SCALE = 1.25 / float(_TOKENS * _DIM)


def _dist_argmin_kernel(x_ref, e_ref, idx_ref, loss_ref):
    e = e_ref[...]
    e2 = jnp.sum(e * e, axis=1)[:, None]
    loss_part = None
    for bb in range(_BM // _T):
        xt = x_ref[bb]
        mm = lax.dot_general(e, xt, (((1,), (0,)), ((), ())),
                             preferred_element_type=jnp.float32)
        x2 = jnp.sum(xt * xt, axis=0)[None, :]
        dist = (x2 - 2.0 * mm) + e2
        m = jnp.min(dist, axis=0, keepdims=True)

        ids = lax.broadcasted_iota(jnp.int32, dist.shape, 0).astype(jnp.float32)
        idx = jnp.min(jnp.where(dist == m, ids, jnp.float32(_N_EMB)), axis=0)
        idx_ref[0, 0, bb * _T:(bb + 1) * _T] = idx.astype(jnp.int32)
        s = jnp.sum(m)
        loss_part = s if loss_part is None else loss_part + s

    @pl.when(pl.program_id(0) == 0)
    def _init():
        loss_ref[...] = jnp.zeros((1, 1), jnp.float32)

    loss_ref[...] += loss_part[None, None]

    @pl.when(pl.program_id(0) == pl.num_programs(0) - 1)
    def _finalize():
        loss_ref[...] = loss_ref[...] * _LOSS_SCALE


_dist_call = pl.pallas_call(
    _dist_argmin_kernel,
    grid=(_G,),
    in_specs=[
        pl.BlockSpec((_BM // _T, _DIM, _T), lambda i: (i, 0, 0)),
        pl.BlockSpec((_N_EMB, _DIM), lambda i: (0, 0)),
    ],
    out_specs=[
        pl.BlockSpec((1, 1, _BM), lambda i: (i, 0, 0)),
        pl.BlockSpec((1, 1), lambda i: (0, 0)),
    ],
    out_shape=[
        jax.ShapeDtypeStruct((_G, 1, _BM), jnp.int32),
        jax.ShapeDtypeStruct((1, 1), jnp.float32),
    ],
)


def _gather_body(idx_hbm, table_hbm, out_hbm, idx_v, rows_v, sem):
    wid = lax.axis_index("s") * 2 + lax.axis_index("c")
    pltpu.sync_copy(idx_hbm.at[wid], idx_v)
    copies = [pltpu.async_copy(table_hbm.at[idx_v.at[j]],
                               rows_v.at[pl.ds(j * _CHUNK, _CHUNK)], sem)
              for j in range(_NCH)]
    for c in copies:
        c.wait()
    g, h = wid // 2, wid % 2
    pltpu.sync_copy(rows_v, out_hbm.at[g, pl.ds(h * _BPW, _BPW)])


_gather_call = pl.kernel(
    _gather_body,
    out_type=jax.ShapeDtypeStruct((_B, _T, _DIM), jnp.float32),
    mesh=plsc.VectorSubcoreMesh(core_axis_name="c", subcore_axis_name="s"),
    compiler_params=pltpu.CompilerParams(use_tc_tiling_on_sc=False),
    scratch_types=[
        pltpu.VMEM((_NCH, _CHUNK), jnp.int32),
        pltpu.VMEM((_BPW, _DIM), jnp.float32),
        pltpu.SemaphoreType.DMA,
    ],
)


@jax.jit
def kernel(inputs, embedding):
    idx3, loss = _dist_call(jnp.swapaxes(inputs, 1, 2), embedding)
    indices = idx3.reshape(_TOKENS)
    quantized_st = _gather_call(indices.reshape(_NW, _NCH, _CHUNK), embedding)
    return quantized_st, loss[0, 0], indices
